# Initial kernel scaffold; baseline (speedup 1.0000x reference)
import jax
import jax.numpy as jnp
from jax.experimental import pallas as pl


def kernel(input_, weight):
    raise NotImplementedError("write your pallas kernel here")


# speedup vs baseline: 1.5467x; 1.5467x over previous
import functools

import jax
import jax.numpy as jnp
from jax import lax
from jax.experimental import pallas as pl
from jax.experimental.pallas import tpu as pltpu
from jax.experimental.pallas import tpu_sc as plsc

_NUM_EMBEDDINGS = 1000000
_EMBEDDING_DIM = 32
_BATCH = 16384
_FIELDS = 26

_INFO = plsc.get_sparse_core_info()
_NC = _INFO.num_cores
_NS = _INFO.num_subcores
_NW = _NC * _NS
_B = _BATCH * _FIELDS
_B_PER_W = _B // _NW
_CHUNK = 1024
_N_CHUNKS = _B_PER_W // _CHUNK


def _gather_body(idx_hbm, table_hbm, out_hbm, idx_v, rows_v, sem):
    wid = lax.axis_index("s") * _NC + lax.axis_index("c")
    base = wid * _B_PER_W

    def step(i, _):
        off = base + i * _CHUNK
        pltpu.sync_copy(idx_hbm.at[pl.ds(off, _CHUNK)], idx_v)
        pltpu.async_copy(table_hbm.at[idx_v], rows_v, sem).wait()
        pltpu.sync_copy(rows_v, out_hbm.at[pl.ds(off, _CHUNK)])
        return ()

    lax.fori_loop(0, _N_CHUNKS, step, ())


@jax.jit
def kernel(input_, weight):
    idx_flat = input_.reshape(_B)
    mesh = plsc.VectorSubcoreMesh(core_axis_name="c", subcore_axis_name="s")
    out = pl.kernel(
        _gather_body,
        out_type=jax.ShapeDtypeStruct((_B, _EMBEDDING_DIM), jnp.float32),
        mesh=mesh,
        scratch_types=[
            pltpu.VMEM((_CHUNK,), jnp.int32),
            pltpu.VMEM((_CHUNK, _EMBEDDING_DIM), jnp.float32),
            pltpu.SemaphoreType.DMA,
        ],
        compiler_params=pltpu.CompilerParams(use_tc_tiling_on_sc=False),
    )(idx_flat, weight)
    return out.reshape(_BATCH, _FIELDS, _EMBEDDING_DIM)


# speedup vs baseline: 1.5749x; 1.0183x over previous
import functools

import jax
import jax.numpy as jnp
from jax import lax
from jax.experimental import pallas as pl
from jax.experimental.pallas import tpu as pltpu
from jax.experimental.pallas import tpu_sc as plsc

_NUM_EMBEDDINGS = 1000000
_EMBEDDING_DIM = 32
_BATCH = 16384
_FIELDS = 26

_INFO = plsc.get_sparse_core_info()
_NC = _INFO.num_cores
_NS = _INFO.num_subcores
_NW = _NC * _NS
_B = _BATCH * _FIELDS
_B_PER_W = _B // _NW
_CHUNK = 832
_N_CHUNKS = _B_PER_W // _CHUNK
_NBUF = 4
_N_GROUPS = _N_CHUNKS // _NBUF


def _gather_body(idx_hbm, table_hbm, out_hbm, *scratch):
    idx_v = scratch[0:_NBUF]
    rows_v = scratch[_NBUF:2 * _NBUF]
    gsem = scratch[2 * _NBUF:3 * _NBUF]
    wsem = scratch[3 * _NBUF:4 * _NBUF]

    wid = lax.axis_index("s") * _NC + lax.axis_index("c")
    base = wid * _B_PER_W

    def start_gather(chunk, b):
        off = base + chunk * _CHUNK
        pltpu.sync_copy(idx_hbm.at[pl.ds(off, _CHUNK)], idx_v[b])
        pltpu.async_copy(table_hbm.at[idx_v[b]], rows_v[b], gsem[b])

    def start_write(chunk, b):
        off = base + chunk * _CHUNK
        pltpu.async_copy(rows_v[b], out_hbm.at[pl.ds(off, _CHUNK)], wsem[b])

    for b in range(_NBUF):
        start_gather(b, b)

    def group(gi, _):
        prev = (gi - 1) * _NBUF
        cur = gi * _NBUF
        for b in range(_NBUF):
            pltpu.make_async_copy(table_hbm.at[idx_v[b]], rows_v[b], gsem[b]).wait()
            start_write(prev + b, b)
        for b in range(_NBUF):
            pltpu.make_async_copy(rows_v[b], out_hbm.at[pl.ds(0, _CHUNK)], wsem[b]).wait()
            start_gather(cur + b, b)
        return ()

    lax.fori_loop(1, _N_GROUPS, group, ())

    last = (_N_GROUPS - 1) * _NBUF
    for b in range(_NBUF):
        pltpu.make_async_copy(table_hbm.at[idx_v[b]], rows_v[b], gsem[b]).wait()
        start_write(last + b, b)
    for b in range(_NBUF):
        pltpu.make_async_copy(rows_v[b], out_hbm.at[pl.ds(0, _CHUNK)], wsem[b]).wait()


@jax.jit
def kernel(input_, weight):
    idx_flat = input_.reshape(_B)
    mesh = plsc.VectorSubcoreMesh(core_axis_name="c", subcore_axis_name="s")
    out = pl.kernel(
        _gather_body,
        out_type=jax.ShapeDtypeStruct((_B, _EMBEDDING_DIM), jnp.float32),
        mesh=mesh,
        scratch_types=(
            [pltpu.VMEM((_CHUNK,), jnp.int32) for _ in range(_NBUF)]
            + [pltpu.VMEM((_CHUNK, _EMBEDDING_DIM), jnp.float32) for _ in range(_NBUF)]
            + [pltpu.SemaphoreType.DMA for _ in range(2 * _NBUF)]
        ),
        compiler_params=pltpu.CompilerParams(use_tc_tiling_on_sc=False),
    )(idx_flat, weight)
    return out.reshape(_BATCH, _FIELDS, _EMBEDDING_DIM)
